# baseline (speedup 1.0000x reference)
import functools

import jax
import jax.numpy as jnp
from jax import lax
from jax.experimental import pallas as pl
from jax.experimental.pallas import tpu as pltpu
from jax.experimental.pallas import tpu_sc as plsc

_N = 32768
_D = 512
_ROWS_BLK = 2048
_NBLK = _N // _ROWS_BLK
_LANES = 16
_BIG = 2 ** 30


def _mv_body(x_ref, w_ref, vx_ref, ix_ref, vn_ref, in_ref):
    i = pl.program_id(0)
    d = jnp.dot(x_ref[...], w_ref[...], preferred_element_type=jnp.float32)
    ri = lax.broadcasted_iota(jnp.int32, (_ROWS_BLK, 1), 0)
    big = jnp.int32(_BIG)
    bmax = jnp.max(d)
    bmin = jnp.min(d)
    vx_ref[i] = bmax
    ix_ref[i] = jnp.min(jnp.where(d == bmax, ri, big)) + i * _ROWS_BLK
    vn_ref[i] = bmin
    in_ref[i] = jnp.min(jnp.where(d == bmin, ri, big)) + i * _ROWS_BLK


def _mv_candidates(feats, w_col):
    sd = jax.ShapeDtypeStruct
    return pl.pallas_call(
        _mv_body,
        grid=(_NBLK,),
        in_specs=[
            pl.BlockSpec((_ROWS_BLK, _D), lambda i: (i, 0)),
            pl.BlockSpec((_D, 1), lambda i: (0, 0)),
        ],
        out_specs=[pl.BlockSpec(memory_space=pltpu.SMEM)] * 4,
        out_shape=[sd((_LANES,), jnp.float32), sd((_LANES,), jnp.int32),
                   sd((_LANES,), jnp.float32), sd((_LANES,), jnp.int32)],
        compiler_params=pltpu.CompilerParams(
            dimension_semantics=("arbitrary",)),
    )(feats, w_col)


_mesh = plsc.VectorSubcoreMesh(core_axis_name="c", subcore_axis_name="s")


@functools.partial(
    pl.kernel,
    mesh=_mesh,
    out_type=jax.ShapeDtypeStruct((2, _D), jnp.float32),
    scratch_types=[
        pltpu.VMEM((_LANES,), jnp.float32),
        pltpu.VMEM((_LANES,), jnp.int32),
        pltpu.VMEM((_LANES,), jnp.float32),
        pltpu.VMEM((_LANES,), jnp.int32),
        pltpu.VMEM((_LANES,), jnp.int32),
        pltpu.VMEM((_LANES, _D), jnp.float32),
        pltpu.SemaphoreType.DMA,
    ],
    compiler_params=pltpu.CompilerParams(needs_layout_passes=False),
)
def _sc_select(vx_hbm, ix_hbm, vn_hbm, in_hbm, feats_hbm, out_hbm,
               vx_v, ix_v, vn_v, in_v, gidx, rows, sem):
    cid = lax.axis_index("c")
    sid = lax.axis_index("s")

    @pl.when(jnp.logical_and(cid == 0, sid == 0))
    def _():
        pltpu.sync_copy(vx_hbm, vx_v)
        pltpu.sync_copy(ix_hbm, ix_v)
        pltpu.sync_copy(vn_hbm, vn_v)
        pltpu.sync_copy(in_hbm, in_v)
        lanes = lax.iota(jnp.int32, _LANES)
        valid = lanes < _NBLK
        big = jnp.int32(_BIG)
        bvx = jnp.where(valid, vx_v[...], -jnp.inf)
        bix = jnp.where(valid, ix_v[...], big)
        bvn = jnp.where(valid, vn_v[...], jnp.inf)
        bni = jnp.where(valid, in_v[...], big)
        for s in (8, 4, 2, 1):
            perm = lanes ^ s
            vx_v[...] = bvx
            ix_v[...] = bix
            vn_v[...] = bvn
            in_v[...] = bni
            ov = plsc.load_gather(vx_v, [perm])
            oi = plsc.load_gather(ix_v, [perm])
            t = (ov > bvx) | ((ov == bvx) & (oi < bix))
            bvx = jnp.where(t, ov, bvx)
            bix = jnp.where(t, oi, bix)
            ov = plsc.load_gather(vn_v, [perm])
            oi = plsc.load_gather(in_v, [perm])
            t = (ov < bvn) | ((ov == bvn) & (oi < bni))
            bvn = jnp.where(t, ov, bvn)
            bni = jnp.where(t, oi, bni)
        gidx[...] = jnp.where(lanes == 0, bix, jnp.where(lanes == 1, bni, 0))
        pltpu.async_copy(feats_hbm.at[gidx], rows, sem).wait()
        pltpu.sync_copy(rows.at[pl.ds(0, 2)], out_hbm)


def kernel(feats, W, b):
    del b
    w_col = (W[0] - W[1]).reshape(_D, 1)
    vx, ix, vn, iN = _mv_candidates(feats, w_col)
    return _sc_select(vx, ix, vn, iN, feats)
